# Initial kernel scaffold; baseline (speedup 1.0000x reference)
import jax
import jax.numpy as jnp
from jax.experimental import pallas as pl


def kernel(scores, gt_boxes, metadata):
    raise NotImplementedError("write your pallas kernel here")


# speedup vs baseline: 1.0345x; 1.0345x over previous
import functools

import jax
import jax.numpy as jnp
import numpy as np
from jax.experimental import pallas as pl
from jax.experimental.pallas import tpu as pltpu

_STRIDE = 16
_NEG_OV = 0.3
_POS_OV = 0.7


def _base_anchors(base_size=16, ratios=(0.5, 1.0, 2.0), scales=(8, 16, 32)):
    base = np.array([1, 1, base_size, base_size], dtype=np.float64) - 1
    w = base[2] - base[0] + 1
    h = base[3] - base[1] + 1
    x_ctr = base[0] + 0.5 * (w - 1)
    y_ctr = base[1] + 0.5 * (h - 1)
    size = w * h
    out = []
    for r in ratios:
        ws = np.round(np.sqrt(size / r))
        hs = np.round(ws * r)
        for s in scales:
            wss = ws * s
            hss = hs * s
            out.append([x_ctr - 0.5 * (wss - 1), y_ctr - 0.5 * (hss - 1),
                        x_ctr + 0.5 * (wss - 1), y_ctr + 0.5 * (hss - 1)])
    return np.array(out, dtype=np.float32)


def _inside_anchors(shape, stride):
    rr, cc = shape
    shift_x = np.arange(0, cc) * stride
    shift_y = np.arange(0, rr) * stride
    sx, sy = np.meshgrid(shift_x, shift_y)
    shifts = np.stack([sx.ravel(), sy.ravel(), sx.ravel(), sy.ravel()],
                      axis=1).astype(np.float32)
    base = _base_anchors(base_size=stride)
    all_anchors = (base.reshape(1, -1, 4) + shifts.reshape(-1, 1, 4)).reshape(-1, 4)
    all_anchors = all_anchors.astype(np.float32)
    mask = ((all_anchors[:, 0] >= 0) & (all_anchors[:, 1] >= 0) &
            (all_anchors[:, 2] < 800.0) & (all_anchors[:, 3] < 800.0))
    return all_anchors[np.where(mask)[0]]


def _anchor_target_body(n_valid, g_valid, a_ref, g_ref, meta_ref,
                        labels_ref, bbox_ref):
    ax1 = a_ref[:, 0:1]
    ay1 = a_ref[:, 1:2]
    ax2 = a_ref[:, 2:3]
    ay2 = a_ref[:, 3:4]
    gx1 = g_ref[0:1, :]
    gy1 = g_ref[1:2, :]
    gx2 = g_ref[2:3, :]
    gy2 = g_ref[3:4, :]

    n_pad = ax1.shape[0]
    g_pad = gx1.shape[1]
    col_i = jax.lax.broadcasted_iota(jnp.int32, (n_pad, g_pad), 1)
    row_i = jax.lax.broadcasted_iota(jnp.int32, (n_pad, g_pad), 0)
    col_valid = col_i < g_valid

    x1 = jnp.maximum(ax1, gx1)
    y1 = jnp.maximum(ay1, gy1)
    x2 = jnp.minimum(ax2, gx2)
    y2 = jnp.minimum(ay2, gy2)
    iw = jnp.clip(x2 - x1 + 1.0, 0.0)
    ih = jnp.clip(y2 - y1 + 1.0, 0.0)
    inter = iw * ih
    area_a = (ax2 - ax1 + 1.0) * (ay2 - ay1 + 1.0)
    area_g = (gx2 - gx1 + 1.0) * (gy2 - gy1 + 1.0)
    ov = inter / (area_a + area_g - inter)
    ov = jnp.where(col_valid, ov, -1.0)

    rowmax = jnp.max(ov, axis=1, keepdims=True)
    row_eq = ov == rowmax
    rowarg = jnp.min(jnp.where(row_eq, col_i, g_pad), axis=1, keepdims=True)

    colmax = jnp.max(ov, axis=0, keepdims=True)
    colarg = jnp.min(jnp.where(ov == colmax, row_i, n_pad), axis=0,
                     keepdims=True)
    pos_gt = jnp.any((row_i == colarg) & col_valid, axis=1, keepdims=True)

    labels = jnp.where(rowmax < _NEG_OV, 0.0, -1.0)
    labels = jnp.where(pos_gt, 1.0, labels)
    labels = jnp.where(rowmax >= _POS_OV, 1.0, labels)

    h = meta_ref[0, 0]
    w = meta_ref[0, 1]
    inside = (ax1 >= 0.0) & (ay1 >= 0.0) & (ax2 < w) & (ay2 < h)
    labels_ref[:, :] = jnp.where(inside, labels, -1.0)

    onehot = ((col_i == rowarg) & col_valid).astype(jnp.float32)
    gw = gx2 - gx1 + 1.0
    gh = gy2 - gy1 + 1.0
    gcx = gx1 + 0.5 * gw
    gcy = gy1 + 0.5 * gh
    sel_cx = jnp.sum(onehot * gcx, axis=1, keepdims=True)
    sel_cy = jnp.sum(onehot * gcy, axis=1, keepdims=True)
    sel_w = jnp.sum(onehot * gw, axis=1, keepdims=True)
    sel_h = jnp.sum(onehot * gh, axis=1, keepdims=True)

    ew = ax2 - ax1 + 1.0
    eh = ay2 - ay1 + 1.0
    ecx = ax1 + 0.5 * ew
    ecy = ay1 + 0.5 * eh
    bbox_ref[:, 0:1] = (sel_cx - ecx) / ew
    bbox_ref[:, 1:2] = (sel_cy - ecy) / eh
    bbox_ref[:, 2:3] = jnp.log(sel_w / ew)
    bbox_ref[:, 3:4] = jnp.log(sel_h / eh)


@functools.partial(jax.jit, static_argnums=(3, 4))
def _run(anchors_pad, gt_t, metadata, n_valid, g_valid):
    n_pad = anchors_pad.shape[0]
    body = functools.partial(_anchor_target_body, n_valid, g_valid)
    labels, bbox = pl.pallas_call(
        body,
        out_shape=[
            jax.ShapeDtypeStruct((n_pad, 1), jnp.float32),
            jax.ShapeDtypeStruct((n_pad, 4), jnp.float32),
        ],
        in_specs=[
            pl.BlockSpec(memory_space=pltpu.VMEM),
            pl.BlockSpec(memory_space=pltpu.VMEM),
            pl.BlockSpec(memory_space=pltpu.SMEM),
        ],
        out_specs=[
            pl.BlockSpec(memory_space=pltpu.VMEM),
            pl.BlockSpec(memory_space=pltpu.VMEM),
        ],
    )(anchors_pad, gt_t, metadata)
    return labels[:n_valid, 0], bbox[:n_valid, :]


def kernel(scores, gt_boxes, metadata):
    rr, cc = scores.shape[1], scores.shape[2]
    anchors_in = _inside_anchors((rr, cc), _STRIDE)
    n_valid = anchors_in.shape[0]
    n_pad = ((n_valid + 127) // 128) * 128
    pad = np.full((n_pad - n_valid, 4), -1.0e6, dtype=np.float32)
    pad[:, 2:] += 1.0
    anchors_pad = jnp.asarray(np.concatenate([anchors_in, pad], axis=0))

    g_valid = gt_boxes.shape[0]
    g_pad = ((g_valid + 127) // 128) * 128
    gt_t = jnp.zeros((8, g_pad), jnp.float32).at[:4, :g_valid].set(gt_boxes.T)

    return _run(anchors_pad, gt_t, metadata, n_valid, g_valid)
